# baseline (speedup 1.0000x reference)
import functools

import jax
import jax.numpy as jnp
from jax import lax
from jax.experimental import pallas as pl
from jax.experimental.pallas import tpu as pltpu
from jax.experimental.pallas import tpu_sc as plsc

BATCH = 1024
HIDDEN = 1024
MEM_VEC = 128
BIN_LEN = 7
PAD_LEN = 8

_POWERS = [float(2 ** (BIN_LEN - 1 - i)) for i in range(BIN_LEN)] + [0.0]

_NC = 2
_NS = 16
_NW = _NC * _NS
_B_PER_W = BATCH // _NW


def _tc_body(x_ref, w2_ref, b2_ref, w_ref, idx_ref):
    w = lax.dot_general(
        x_ref[...], w2_ref[...],
        dimension_numbers=(((1,), (1,)), ((), ())),
        preferred_element_type=jnp.float32,
    )
    w = jnp.maximum(w + b2_ref[...], 0.0)
    w_ref[...] = w
    bits = (w > 0.5).astype(jnp.int32)
    col = lax.broadcasted_iota(jnp.int32, (1, PAD_LEN), 1)
    powers = jnp.where(
        col < BIN_LEN, lax.shift_left(1, jnp.maximum(BIN_LEN - 1 - col, 0)), 0
    )
    idx_ref[...] = jnp.sum(bits * powers, axis=1, keepdims=True)


_tc_call = pl.pallas_call(
    _tc_body,
    out_shape=[
        jax.ShapeDtypeStruct((BATCH, PAD_LEN), jnp.float32),
        jax.ShapeDtypeStruct((BATCH, 1), jnp.int32),
    ],
)


@functools.cache
def _make_sc_gather():
    mesh = plsc.VectorSubcoreMesh(
        core_axis_name="c", subcore_axis_name="s",
        num_cores=_NC, num_subcores=_NS,
    )

    @functools.partial(
        pl.kernel,
        mesh=mesh,
        out_type=jax.ShapeDtypeStruct((BATCH, MEM_VEC), jnp.float32),
        scratch_types=[
            pltpu.VMEM((_B_PER_W,), jnp.int32),
            pltpu.VMEM((_B_PER_W, MEM_VEC), jnp.float32),
            pltpu.SemaphoreType.DMA,
        ],
    )
    def _sc_gather(idx_hbm, table_hbm, out_hbm, idx_v, rows_v, sem):
        wid = lax.axis_index("s") * _NC + lax.axis_index("c")
        base = wid * _B_PER_W
        pltpu.sync_copy(idx_hbm.at[pl.ds(base, _B_PER_W)], idx_v)
        pltpu.async_copy(table_hbm.at[idx_v], rows_v, sem).wait()
        pltpu.sync_copy(rows_v, out_hbm.at[pl.ds(base, _B_PER_W)])

    return _sc_gather


def kernel(x, previous_state, W2, b2, memory):
    W2p = jnp.zeros((PAD_LEN, HIDDEN), jnp.float32).at[:BIN_LEN].set(W2)
    b2p = jnp.full((1, PAD_LEN), -1.0, jnp.float32).at[0, :BIN_LEN].set(b2)
    w_pad, idx2d = _tc_call(x, W2p, b2p)
    idx = idx2d.reshape(BATCH)
    memory_read = _make_sc_gather()(idx, memory)
    return memory_read, w_pad[:, :BIN_LEN]


# speedup vs baseline: 1.1866x; 1.1866x over previous
import functools

import jax
import jax.numpy as jnp
from jax import lax
from jax.experimental import pallas as pl
from jax.experimental.pallas import tpu as pltpu
from jax.experimental.pallas import tpu_sc as plsc

BATCH = 1024
HIDDEN = 1024
MEM_VEC = 128
BIN_LEN = 7

_NC = 2
_NS = 16
_NW = _NC * _NS
_B_PER_W = BATCH // _NW
_IDX_ROWS = 8
_IDX_COLS = BATCH // _IDX_ROWS
_CHUNKS_PER_ROW = _IDX_COLS // _B_PER_W


def _tc_body(x_ref, w2_ref, b2_ref, w_ref, idx_ref):
    w = lax.dot_general(
        x_ref[...], w2_ref[...],
        dimension_numbers=(((1,), (1,)), ((), ())),
        preferred_element_type=jnp.float32,
    )
    w = jnp.maximum(w + b2_ref[...], 0.0)
    w_ref[...] = w
    bits = (w > 0.5).astype(jnp.float32)
    col = lax.broadcasted_iota(jnp.int32, (1, BIN_LEN), 1)
    powers = lax.shift_left(1, BIN_LEN - 1 - col).astype(jnp.float32)
    idx_row = lax.dot_general(
        powers, bits,
        dimension_numbers=(((1,), (1,)), ((), ())),
        preferred_element_type=jnp.float32,
    )
    idx_ref[...] = jnp.reshape(idx_row.astype(jnp.int32), (_IDX_ROWS, _IDX_COLS))


_tc_call = pl.pallas_call(
    _tc_body,
    out_shape=[
        jax.ShapeDtypeStruct((BATCH, BIN_LEN), jnp.float32),
        jax.ShapeDtypeStruct((_IDX_ROWS, _IDX_COLS), jnp.int32),
    ],
)


@functools.cache
def _make_sc_gather():
    mesh = plsc.VectorSubcoreMesh(
        core_axis_name="c", subcore_axis_name="s",
        num_cores=_NC, num_subcores=_NS,
    )

    @functools.partial(
        pl.kernel,
        mesh=mesh,
        out_type=jax.ShapeDtypeStruct((BATCH, MEM_VEC), jnp.float32),
        scratch_types=[
            pltpu.VMEM((1, _IDX_COLS), jnp.int32),
            pltpu.VMEM((_B_PER_W, MEM_VEC), jnp.float32),
            pltpu.SemaphoreType.DMA,
        ],
    )
    def _sc_gather(idx_hbm, table_hbm, out_hbm, idx_v, rows_v, sem):
        wid = lax.axis_index("s") * _NC + lax.axis_index("c")
        row = wid // _CHUNKS_PER_ROW
        col = (wid % _CHUNKS_PER_ROW) * _B_PER_W
        pltpu.sync_copy(idx_hbm.at[pl.ds(row, 1)], idx_v)
        pltpu.async_copy(
            table_hbm.at[idx_v.at[0, pl.ds(col, _B_PER_W)]], rows_v, sem
        ).wait()
        pltpu.sync_copy(rows_v, out_hbm.at[pl.ds(wid * _B_PER_W, _B_PER_W)])

    return _sc_gather


def kernel(x, previous_state, W2, b2, memory):
    w, idx8 = _tc_call(x, W2, b2.reshape(1, BIN_LEN))
    memory_read = _make_sc_gather()(idx8, memory)
    return memory_read, w
